# baseline (speedup 1.0000x reference)
import jax
import jax.numpy as jnp
from jax.experimental import pallas as pl
from jax.experimental.pallas import tpu as pltpu

_B = 64
_BB = 2
_NBLK = _B // _BB


def _berhu_body(x_ref, y_ref, out_ref, acc_ref, vacc_ref, cache_ref):
    p = pl.program_id(0)
    j = pl.program_id(1)

    @pl.when(p == 0)
    def _():
        d = jnp.abs(x_ref[...] - y_ref[...])

        @pl.when(j == 0)
        def _():
            acc_ref[0] = 0.0
            acc_ref[1] = 0.0

        acc_ref[0] = jnp.maximum(acc_ref[0], jnp.max(d))
        acc_ref[1] += jnp.sum(d)
        cache_ref[pl.ds(j * _BB, _BB)] = d.astype(jnp.bfloat16)

    @pl.when(p == 1)
    def _():
        @pl.when(j == 0)
        def _():
            vacc_ref[...] = jnp.zeros_like(vacc_ref)

        cb = jnp.maximum(acc_ref[0] * 0.2, 0.0001).astype(jnp.bfloat16)
        d = cache_ref[pl.ds(j * _BB, _BB)].reshape(_BB * 512, 512)
        u = jnp.maximum(d - cb, jnp.bfloat16(0.0))
        ones = jnp.ones((_BB * 512,), jnp.bfloat16)
        sq = jax.lax.dot_general(ones, u * u, (((0,), (0,)), ((), ())),
                                 preferred_element_type=jnp.float32)
        vacc_ref[0, :] += sq

        @pl.when(j == _NBLK - 1)
        def _():
            c32 = cb.astype(jnp.float32)
            out_ref[0] = acc_ref[1] + jnp.sum(vacc_ref[0, :]) / (2.0 * c32)


def kernel(prediction, target):
    spec = pl.BlockSpec(
        (_BB, 1, 512, 512), lambda p, j: (jnp.where(p == 0, j, 0), 0, 0, 0))
    out = pl.pallas_call(
        _berhu_body,
        grid=(2, _NBLK),
        in_specs=[spec, spec],
        out_specs=pl.BlockSpec(memory_space=pltpu.SMEM),
        out_shape=jax.ShapeDtypeStruct((1,), jnp.float32),
        scratch_shapes=[
            pltpu.SMEM((4,), jnp.float32),
            pltpu.VMEM((1, 512), jnp.float32),
            pltpu.VMEM((_B, 1, 512, 512), jnp.bfloat16),
        ],
        compiler_params=pltpu.CompilerParams(
            dimension_semantics=("arbitrary", "arbitrary"),
            vmem_limit_bytes=64 * 1024 * 1024,
        ),
    )(prediction, target)
    return out.reshape(())


# speedup vs baseline: 1.1943x; 1.1943x over previous
import jax
import jax.numpy as jnp
from jax.experimental import pallas as pl
from jax.experimental.pallas import tpu as pltpu

_B = 64
_BB = 4
_NBLK = _B // _BB


def _colsum_usq(d, cb):
    u = jnp.maximum(d - cb, jnp.bfloat16(0.0))
    ones = jnp.ones((_BB * 512,), jnp.bfloat16)
    return jax.lax.dot_general(ones, u * u, (((0,), (0,)), ((), ())),
                               preferred_element_type=jnp.float32)


def _berhu_body(x_ref, y_ref, out_ref, acc_ref, cj_ref, vacc_ref, qspec_ref,
                cache_ref):
    p = pl.program_id(0)
    j = pl.program_id(1)

    @pl.when(p == 0)
    def _():
        d = jnp.abs(x_ref[...] - y_ref[...])

        @pl.when(j == 0)
        def _():
            acc_ref[0] = 0.0
            acc_ref[1] = 0.0

        acc_ref[0] = jnp.maximum(acc_ref[0], jnp.max(d))
        acc_ref[1] += jnp.sum(d)
        db = d.astype(jnp.bfloat16)
        cache_ref[pl.ds(j * _BB, _BB)] = db
        cj = jnp.maximum(acc_ref[0] * 0.2, 0.0001)
        cj_ref[j] = cj
        qspec_ref[j, :] = _colsum_usq(
            db.reshape(_BB * 512, 512), cj.astype(jnp.bfloat16))

    @pl.when(p == 1)
    def _():
        @pl.when(j == 0)
        def _():
            vacc_ref[...] = jnp.zeros_like(vacc_ref)

        cf = jnp.maximum(acc_ref[0] * 0.2, 0.0001)

        @pl.when(cj_ref[j] == cf)
        def _():
            vacc_ref[0, :] += qspec_ref[j, :]

        @pl.when(cj_ref[j] != cf)
        def _():
            d = cache_ref[pl.ds(j * _BB, _BB)].reshape(_BB * 512, 512)
            vacc_ref[0, :] += _colsum_usq(d, cf.astype(jnp.bfloat16))

        @pl.when(j == _NBLK - 1)
        def _():
            c32 = cf.astype(jnp.bfloat16).astype(jnp.float32)
            out_ref[0] = acc_ref[1] + jnp.sum(vacc_ref[0, :]) / (2.0 * c32)


def kernel(prediction, target):
    spec = pl.BlockSpec(
        (_BB, 1, 512, 512), lambda p, j: (jnp.where(p == 0, j, 0), 0, 0, 0))
    out = pl.pallas_call(
        _berhu_body,
        grid=(2, _NBLK),
        in_specs=[spec, spec],
        out_specs=pl.BlockSpec(memory_space=pltpu.SMEM),
        out_shape=jax.ShapeDtypeStruct((1,), jnp.float32),
        scratch_shapes=[
            pltpu.SMEM((4,), jnp.float32),
            pltpu.SMEM((_NBLK,), jnp.float32),
            pltpu.VMEM((1, 512), jnp.float32),
            pltpu.VMEM((_NBLK, 512), jnp.float32),
            pltpu.VMEM((_B, 1, 512, 512), jnp.bfloat16),
        ],
        compiler_params=pltpu.CompilerParams(
            dimension_semantics=("arbitrary", "arbitrary"),
            vmem_limit_bytes=64 * 1024 * 1024,
        ),
    )(prediction, target)
    return out.reshape(())
